# baseline (speedup 1.0000x reference)
import functools

import jax
import jax.numpy as jnp
from jax import lax
from jax.experimental import pallas as pl
from jax.experimental.pallas import tpu as pltpu
from jax.experimental.pallas import tpu_sc as plsc

NC, NS, LANES = 2, 16, 16
NW = NC * NS


def _sc_relayout(emb_t, K, M):
    C = 1792
    NCHK = (M // 128) * 128 // C
    ITER = (NCHK + NW - 1) // NW
    GC = C // LANES

    mesh = plsc.VectorSubcoreMesh(core_axis_name="c", subcore_axis_name="s")

    @functools.partial(
        pl.kernel,
        out_type=jax.ShapeDtypeStruct((M * K,), jnp.float32),
        mesh=mesh,
        scratch_types=[
            pltpu.VMEM((K, C), jnp.float32),
            pltpu.VMEM((K, C), jnp.float32),
            pltpu.VMEM((C * K,), jnp.float32),
            pltpu.VMEM((C * K,), jnp.float32),
            pltpu.SemaphoreType.DMA,
            pltpu.SemaphoreType.DMA,
        ],
        compiler_params=pltpu.CompilerParams(
            use_tc_tiling_on_sc=True, needs_layout_passes=False),
    )
    def k(src_hbm, out_hbm, in_a, in_b, out_a, out_b, sem_a, sem_b):
        wid = lax.axis_index("s") * NC + lax.axis_index("c")
        inbufs = (in_a, in_b)
        outbufs = (out_a, out_b)
        sems = (sem_a, sem_b)
        scatter_idx = [lax.iota(jnp.int32, LANES) * K + d for d in range(K)]

        def cid_of(p):
            return jnp.minimum(p * NW + wid, NCHK - 1)

        def fire(p):
            return [pltpu.async_copy(
                src_hbm.at[:, pl.ds(cid_of(p) * C, C)],
                inbufs[p % 2], sems[p % 2])]

        pend = {0: fire(0)}
        for p in range(ITER):
            if p + 1 < ITER:
                pend[p + 1] = fire(p + 1)
            for cp in pend.pop(p):
                cp.wait()
            inb = inbufs[p % 2]
            outb = outbufs[p % 2]

            @plsc.parallel_loop(0, GC)
            def grp(g):
                owin = outb.at[pl.ds(g * (LANES * K), LANES * K)]
                for d in range(K):
                    v = inb[d, pl.ds(g * LANES, LANES)]
                    plsc.store_scatter(owin, [scatter_idx[d]], v)

            pltpu.sync_copy(outb, out_hbm.at[pl.ds(cid_of(p) * (C * K), C * K)])

    return k(emb_t)


def _sc_gather_fm(feat_t, emb, bias_flat, B, F, D):
    EPW = B // NW
    CH = 64
    NCH = EPW // CH
    GRP = CH // LANES

    mesh = plsc.VectorSubcoreMesh(core_axis_name="c", subcore_axis_name="s")

    @functools.partial(
        pl.kernel,
        out_type=(
            jax.ShapeDtypeStruct((B, D), jnp.float32),
            jax.ShapeDtypeStruct((B,), jnp.float32),
        ),
        mesh=mesh,
        scratch_types=[
            pltpu.VMEM((F, EPW), jnp.int32),
            pltpu.VMEM((2, F, CH, D), jnp.float32),
            pltpu.VMEM((2, F, CH), jnp.float32),
            pltpu.VMEM((CH, D), jnp.float32),
            pltpu.VMEM((CH,), jnp.float32),
            pltpu.SemaphoreType.DMA,
            pltpu.SemaphoreType.DMA,
            pltpu.SemaphoreType.DMA,
            pltpu.SemaphoreType.DMA,
        ],
        compiler_params=pltpu.CompilerParams(use_tc_tiling_on_sc=False),
    )
    def k(feat_hbm, emb_hbm, bias_hbm, fm_hbm, fb_hbm,
          idx_t, rows3, bias3, fm_v, fb_v, sem_r0, sem_r1, sem_b0, sem_b1):
        wid = lax.axis_index("s") * NC + lax.axis_index("c")
        ebase = wid * EPW
        pltpu.sync_copy(feat_hbm.at[:, pl.ds(ebase, EPW)], idx_t)
        sems_r = (sem_r0, sem_r1)
        sems_b = (sem_b0, sem_b1)

        def fire(c):
            bi = c % 2
            cps = []
            for f in range(F):
                sl = pl.ds(c * CH, CH)
                cps.append(pltpu.async_copy(
                    emb_hbm.at[idx_t.at[f, sl]], rows3.at[bi, f], sems_r[bi]))
                cps.append(pltpu.async_copy(
                    bias_hbm.at[idx_t.at[f, sl]], bias3.at[bi, f], sems_b[bi]))
            return cps

        pending = {0: fire(0)}
        for c in range(NCH):
            if c + 1 < NCH:
                pending[c + 1] = fire(c + 1)
            for cp in pending.pop(c):
                cp.wait()
            bi = c % 2

            @plsc.parallel_loop(0, CH, unroll=2)
            def elem(i):
                a0 = rows3[bi, 0, i]
                a1 = rows3[bi, 1, i]
                s0 = a0 * a0
                s1 = a1 * a1
                for f in range(2, F, 2):
                    v0 = rows3[bi, f, i]
                    a0 = a0 + v0
                    s0 = s0 + v0 * v0
                    v1 = rows3[bi, f + 1, i]
                    a1 = a1 + v1
                    s1 = s1 + v1 * v1
                acc = a0 + a1
                fm_v[i] = 0.5 * (acc * acc - (s0 + s1))

            for g in range(GRP):
                sl = pl.ds(g * LANES, LANES)
                b0 = bias3[bi, 0, sl]
                b1_ = bias3[bi, 1, sl]
                for f in range(2, F, 2):
                    b0 = b0 + bias3[bi, f, sl]
                    b1_ = b1_ + bias3[bi, f + 1, sl]
                fb_v[sl] = b0 + b1_

            pltpu.sync_copy(fm_v, fm_hbm.at[pl.ds(ebase + c * CH, CH), :])
            pltpu.sync_copy(fb_v, fb_hbm.at[pl.ds(ebase + c * CH, CH)])

    return k(feat_t, emb, bias_flat)


def _tc_mlp_fat(fm_fat, W1b, b1f, W2b, b2f, Wpb, cf):
    R = fm_fat.shape[0]
    BLK = 512

    def body(x_ref, W1_ref, b1_ref, W2_ref, b2_ref, Wp_ref, cf_ref, o_ref):
        x = x_ref[...]
        h = jnp.maximum(
            jnp.dot(x, W1_ref[...], preferred_element_type=jnp.float32)
            + b1_ref[...], 0.0)
        h = jnp.maximum(
            jnp.dot(h, W2_ref[...], preferred_element_type=jnp.float32)
            + b2_ref[...], 0.0)
        o_ref[...] = (jnp.dot(h, Wp_ref[...], preferred_element_type=jnp.float32)
                      + cf_ref[...])

    full = lambda a: pl.BlockSpec(a.shape, lambda i: (0, 0))
    return pl.pallas_call(
        body,
        grid=(R // BLK,),
        in_specs=[
            pl.BlockSpec((BLK, 128), lambda i: (i, 0)),
            full(W1b), full(b1f), full(W2b), full(b2f), full(Wpb), full(cf),
        ],
        out_specs=pl.BlockSpec((BLK, 8), lambda i: (i, 0)),
        out_shape=jax.ShapeDtypeStruct((R, 8), jnp.float32),
    )(fm_fat, W1b, b1f, W2b, b2f, Wpb, cf)


def kernel(features, labels, emb, bias_table, W_bias, W1, b1, W2, b2, Wp, bp):
    B, F = features.shape
    M, D = emb.shape
    bias_flat = bias_table.reshape(M)
    emb_flat = _sc_relayout(emb.T, D, M)
    covered = (1792 * 558) * D
    tail = emb[covered // D:, :].reshape(-1)
    emb_flat = jax.lax.dynamic_update_slice(emb_flat, tail, (covered,))
    emb_rm = emb_flat.reshape(M, D)
    fm, fbias = _sc_gather_fm(features.T, emb_rm, bias_flat, B, F, D)

    eye8 = jnp.eye(8, dtype=jnp.float32)
    W1b = jnp.kron(eye8, W1)
    W2b = jnp.kron(eye8, W2)
    Wpb = jnp.kron(eye8, Wp)
    b1f = jnp.tile(b1, 8).reshape(1, -1)
    b2f = jnp.tile(b2, 8).reshape(1, -1)
    cf = (bp[0] + W_bias[0, 0]).reshape(1, 1)

    fm_fat = fm.reshape(B // 8, 128)
    out_fat = _tc_mlp_fat(fm_fat, W1b, b1f, W2b, b2f, Wpb, cf)
    return out_fat.reshape(B, 1) + fbias.reshape(B, 1)


# speedup vs baseline: 1.0129x; 1.0129x over previous
import functools

import jax
import jax.numpy as jnp
from jax import lax
from jax.experimental import pallas as pl
from jax.experimental.pallas import tpu as pltpu
from jax.experimental.pallas import tpu_sc as plsc

NC, NS, LANES = 2, 16, 16
NW = NC * NS


def _sc_relayout(emb_t, K, M):
    C = 1792
    NCHK = (M // 128) * 128 // C
    ITER = (NCHK + NW - 1) // NW
    GC = C // LANES

    mesh = plsc.VectorSubcoreMesh(core_axis_name="c", subcore_axis_name="s")

    @functools.partial(
        pl.kernel,
        out_type=jax.ShapeDtypeStruct((M * K,), jnp.float32),
        mesh=mesh,
        scratch_types=[
            pltpu.VMEM((K, C), jnp.float32),
            pltpu.VMEM((K, C), jnp.float32),
            pltpu.VMEM((C * K,), jnp.float32),
            pltpu.VMEM((C * K,), jnp.float32),
            pltpu.SemaphoreType.DMA,
            pltpu.SemaphoreType.DMA,
        ],
        compiler_params=pltpu.CompilerParams(
            use_tc_tiling_on_sc=True, needs_layout_passes=False),
    )
    def k(src_hbm, out_hbm, in_a, in_b, out_a, out_b, sem_a, sem_b):
        wid = lax.axis_index("s") * NC + lax.axis_index("c")
        inbufs = (in_a, in_b)
        outbufs = (out_a, out_b)
        sems = (sem_a, sem_b)
        scatter_idx = [lax.iota(jnp.int32, LANES) * K + d for d in range(K)]

        def cid_of(p):
            return jnp.minimum(p * NW + wid, NCHK - 1)

        def fire(p):
            return [pltpu.async_copy(
                src_hbm.at[:, pl.ds(cid_of(p) * C, C)],
                inbufs[p % 2], sems[p % 2])]

        pend = {0: fire(0)}
        for p in range(ITER):
            if p + 1 < ITER:
                pend[p + 1] = fire(p + 1)
            for cp in pend.pop(p):
                cp.wait()
            inb = inbufs[p % 2]
            outb = outbufs[p % 2]

            @plsc.parallel_loop(0, GC)
            def grp(g):
                owin = outb.at[pl.ds(g * (LANES * K), LANES * K)]
                for d in range(K):
                    v = inb[d, pl.ds(g * LANES, LANES)]
                    plsc.store_scatter(owin, [scatter_idx[d]], v)

            pltpu.sync_copy(outb, out_hbm.at[pl.ds(cid_of(p) * (C * K), C * K)])

    return k(emb_t)


def _sc_gather_fm(feat_t, emb, bias_flat, B, F, D):
    EPW = B // NW
    CH = 64
    NCH = EPW // CH
    GRP = CH // LANES

    mesh = plsc.VectorSubcoreMesh(core_axis_name="c", subcore_axis_name="s")

    @functools.partial(
        pl.kernel,
        out_type=(
            jax.ShapeDtypeStruct((B, D), jnp.float32),
            jax.ShapeDtypeStruct((B,), jnp.float32),
        ),
        mesh=mesh,
        scratch_types=[
            pltpu.VMEM((F, EPW), jnp.int32),
            pltpu.VMEM((2, F, CH, D), jnp.float32),
            pltpu.VMEM((2, F, CH), jnp.float32),
            pltpu.VMEM((CH, D), jnp.float32),
            pltpu.VMEM((CH,), jnp.float32),
            pltpu.SemaphoreType.DMA,
            pltpu.SemaphoreType.DMA,
            pltpu.SemaphoreType.DMA,
            pltpu.SemaphoreType.DMA,
        ],
        compiler_params=pltpu.CompilerParams(use_tc_tiling_on_sc=False),
    )
    def k(feat_hbm, emb_hbm, bias_hbm, fm_hbm, fb_hbm,
          idx_t, rows3, bias3, fm_v, fb_v, sem_r0, sem_r1, sem_b0, sem_b1):
        wid = lax.axis_index("s") * NC + lax.axis_index("c")
        ebase = wid * EPW
        pltpu.sync_copy(feat_hbm.at[:, pl.ds(ebase, EPW)], idx_t)
        sems_r = (sem_r0, sem_r1)
        sems_b = (sem_b0, sem_b1)

        def fire(c):
            bi = c % 2
            cps = []
            for f in range(F):
                sl = pl.ds(c * CH, CH)
                cps.append(pltpu.async_copy(
                    emb_hbm.at[idx_t.at[f, sl]], rows3.at[bi, f], sems_r[bi]))
                cps.append(pltpu.async_copy(
                    bias_hbm.at[idx_t.at[f, sl]], bias3.at[bi, f], sems_b[bi]))
            return cps

        pending = {0: fire(0)}
        for c in range(NCH):
            if c + 1 < NCH:
                pending[c + 1] = fire(c + 1)
            for cp in pending.pop(c):
                cp.wait()
            bi = c % 2

            @plsc.parallel_loop(0, CH)
            def elem(i):
                a0 = rows3[bi, 0, i]
                a1 = rows3[bi, 1, i]
                s0 = a0 * a0
                s1 = a1 * a1
                for f in range(2, F, 2):
                    v0 = rows3[bi, f, i]
                    a0 = a0 + v0
                    s0 = s0 + v0 * v0
                    v1 = rows3[bi, f + 1, i]
                    a1 = a1 + v1
                    s1 = s1 + v1 * v1
                acc = a0 + a1
                fm_v[i] = 0.5 * (acc * acc - (s0 + s1))

            for g in range(GRP):
                sl = pl.ds(g * LANES, LANES)
                b0 = bias3[bi, 0, sl]
                b1_ = bias3[bi, 1, sl]
                for f in range(2, F, 2):
                    b0 = b0 + bias3[bi, f, sl]
                    b1_ = b1_ + bias3[bi, f + 1, sl]
                fb_v[sl] = b0 + b1_

            pltpu.sync_copy(fm_v, fm_hbm.at[pl.ds(ebase + c * CH, CH), :])
            pltpu.sync_copy(fb_v, fb_hbm.at[pl.ds(ebase + c * CH, CH)])

    return k(feat_t, emb, bias_flat)


def _tc_mlp_fat(fm_fat, W1b, b1f, W2b, b2f, Wpb, cf):
    R = fm_fat.shape[0]
    BLK = 512

    def body(x_ref, W1_ref, b1_ref, W2_ref, b2_ref, Wp_ref, cf_ref, o_ref):
        x = x_ref[...]
        h = jnp.maximum(
            jnp.dot(x, W1_ref[...], preferred_element_type=jnp.float32)
            + b1_ref[...], 0.0)
        h = jnp.maximum(
            jnp.dot(h, W2_ref[...], preferred_element_type=jnp.float32)
            + b2_ref[...], 0.0)
        o_ref[...] = (jnp.dot(h, Wp_ref[...], preferred_element_type=jnp.float32)
                      + cf_ref[...])

    full = lambda a: pl.BlockSpec(a.shape, lambda i: (0, 0))
    return pl.pallas_call(
        body,
        grid=(R // BLK,),
        in_specs=[
            pl.BlockSpec((BLK, 128), lambda i: (i, 0)),
            full(W1b), full(b1f), full(W2b), full(b2f), full(Wpb), full(cf),
        ],
        out_specs=pl.BlockSpec((BLK, 8), lambda i: (i, 0)),
        out_shape=jax.ShapeDtypeStruct((R, 8), jnp.float32),
    )(fm_fat, W1b, b1f, W2b, b2f, Wpb, cf)


def kernel(features, labels, emb, bias_table, W_bias, W1, b1, W2, b2, Wp, bp):
    B, F = features.shape
    M, D = emb.shape
    bias_flat = bias_table.reshape(M)
    emb_flat = _sc_relayout(emb.T, D, M)
    covered = (1792 * 558) * D
    tail = emb[covered // D:, :].reshape(-1)
    emb_flat = jax.lax.dynamic_update_slice(emb_flat, tail, (covered,))
    emb_rm = emb_flat.reshape(M, D)
    fm, fbias = _sc_gather_fm(features.T, emb_rm, bias_flat, B, F, D)

    eye8 = jnp.eye(8, dtype=jnp.float32)
    W1b = jnp.kron(eye8, W1)
    W2b = jnp.kron(eye8, W2)
    Wpb = jnp.kron(eye8, Wp)
    b1f = jnp.tile(b1, 8).reshape(1, -1)
    b2f = jnp.tile(b2, 8).reshape(1, -1)
    cf = (bp[0] + W_bias[0, 0]).reshape(1, 1)

    fm_fat = fm.reshape(B // 8, 128)
    out_fat = _tc_mlp_fat(fm_fat, W1b, b1f, W2b, b2f, Wpb, cf)
    return out_fat.reshape(B, 1) + fbias.reshape(B, 1)
